# baseline (speedup 1.0000x reference)
import jax
import jax.numpy as jnp
from jax.experimental import pallas as pl

_TILE = 4096


def _conv1x1_kernel(x_ref, y_ref, w1_ref, w2_ref, b_ref, o_ref):
    xb = x_ref[0]
    yb = y_ref[0]
    acc = jnp.dot(w1_ref[...], xb, preferred_element_type=jnp.float32)
    acc = acc + jnp.dot(w2_ref[...], yb, preferred_element_type=jnp.float32)
    o_ref[0] = acc + b_ref[...]


def kernel(x, y, W_proj, b_proj):
    B, C, H, W = x.shape
    HW = H * W
    x3 = x.reshape(B, C, HW)
    y3 = y.reshape(B, C, HW)
    w1 = W_proj[:, :C]
    w2 = W_proj[:, C:]
    b2 = b_proj.reshape(C, 1)

    out = pl.pallas_call(
        _conv1x1_kernel,
        grid=(B, HW // _TILE),
        in_specs=[
            pl.BlockSpec((1, C, _TILE), lambda b, j: (b, 0, j)),
            pl.BlockSpec((1, C, _TILE), lambda b, j: (b, 0, j)),
            pl.BlockSpec((C, C), lambda b, j: (0, 0)),
            pl.BlockSpec((C, C), lambda b, j: (0, 0)),
            pl.BlockSpec((C, 1), lambda b, j: (0, 0)),
        ],
        out_specs=pl.BlockSpec((1, C, _TILE), lambda b, j: (b, 0, j)),
        out_shape=jax.ShapeDtypeStruct((B, C, HW), jnp.float32),
    )(x3, y3, w1, w2, b2)
    return out.reshape(B, C, H, W)


# speedup vs baseline: 1.0284x; 1.0284x over previous
import jax
import jax.numpy as jnp
from jax.experimental import pallas as pl

_TILE = 16384


def _conv1x1_kernel(x_ref, y_ref, w1_ref, w2_ref, b_ref, o_ref):
    xb = x_ref[0]
    yb = y_ref[0]
    acc = jnp.dot(w1_ref[...], xb, preferred_element_type=jnp.float32)
    acc = acc + jnp.dot(w2_ref[...], yb, preferred_element_type=jnp.float32)
    o_ref[0] = acc + b_ref[...]


def kernel(x, y, W_proj, b_proj):
    B, C, H, W = x.shape
    HW = H * W
    x3 = x.reshape(B, C, HW)
    y3 = y.reshape(B, C, HW)
    w1 = W_proj[:, :C]
    w2 = W_proj[:, C:]
    b2 = b_proj.reshape(C, 1)

    out = pl.pallas_call(
        _conv1x1_kernel,
        grid=(B, HW // _TILE),
        in_specs=[
            pl.BlockSpec((1, C, _TILE), lambda b, j: (b, 0, j)),
            pl.BlockSpec((1, C, _TILE), lambda b, j: (b, 0, j)),
            pl.BlockSpec((C, C), lambda b, j: (0, 0)),
            pl.BlockSpec((C, C), lambda b, j: (0, 0)),
            pl.BlockSpec((C, 1), lambda b, j: (0, 0)),
        ],
        out_specs=pl.BlockSpec((1, C, _TILE), lambda b, j: (b, 0, j)),
        out_shape=jax.ShapeDtypeStruct((B, C, HW), jnp.float32),
    )(x3, y3, w1, w2, b2)
    return out.reshape(B, C, H, W)


# speedup vs baseline: 2.1472x; 2.0879x over previous
import jax
import jax.numpy as jnp
from jax.experimental import pallas as pl

_HB = 32


def _conv1x1_kernel(x_ref, y_ref, w1_ref, w2_ref, b_ref, o_ref):
    w1 = w1_ref[...]
    w2 = w2_ref[...]
    b = b_ref[...]
    for h in range(_HB):
        xs = x_ref[0, :, h, :]
        ys = y_ref[0, :, h, :]
        acc = jnp.dot(w1, xs, preferred_element_type=jnp.float32)
        acc = acc + jnp.dot(w2, ys, preferred_element_type=jnp.float32)
        o_ref[0, :, h, :] = acc + b


def kernel(x, y, W_proj, b_proj):
    B, C, H, W = x.shape
    w1 = W_proj[:, :C]
    w2 = W_proj[:, C:]
    b2 = b_proj.reshape(C, 1)

    return pl.pallas_call(
        _conv1x1_kernel,
        grid=(B, H // _HB),
        in_specs=[
            pl.BlockSpec((1, C, _HB, W), lambda b, j: (b, 0, j, 0)),
            pl.BlockSpec((1, C, _HB, W), lambda b, j: (b, 0, j, 0)),
            pl.BlockSpec((C, C), lambda b, j: (0, 0)),
            pl.BlockSpec((C, C), lambda b, j: (0, 0)),
            pl.BlockSpec((C, 1), lambda b, j: (0, 0)),
        ],
        out_specs=pl.BlockSpec((1, C, _HB, W), lambda b, j: (b, 0, j, 0)),
        out_shape=jax.ShapeDtypeStruct((B, C, H, W), jnp.float32),
    )(x, y, w1, w2, b2)


# speedup vs baseline: 3.3653x; 1.5673x over previous
import jax
import jax.numpy as jnp
from jax.experimental import pallas as pl

_HB = 32


def _conv1x1_kernel(x_ref, y_ref, w1_ref, w2_ref, b_ref, o_ref):
    w1 = w1_ref[...]
    w2 = w2_ref[...]
    b = b_ref[...]
    C = w1.shape[0]
    xs = x_ref[0].reshape(C, _HB * 128)
    ys = y_ref[0].reshape(C, _HB * 128)
    acc = jnp.dot(w1, xs, preferred_element_type=jnp.float32)
    acc = acc + jnp.dot(w2, ys, preferred_element_type=jnp.float32)
    o_ref[0] = (acc + b).reshape(C, _HB, 128)


def kernel(x, y, W_proj, b_proj):
    B, C, H, W = x.shape
    w1 = W_proj[:, :C]
    w2 = W_proj[:, C:]
    b2 = b_proj.reshape(C, 1)

    return pl.pallas_call(
        _conv1x1_kernel,
        grid=(B, H // _HB),
        in_specs=[
            pl.BlockSpec((1, C, _HB, W), lambda b, j: (b, 0, j, 0)),
            pl.BlockSpec((1, C, _HB, W), lambda b, j: (b, 0, j, 0)),
            pl.BlockSpec((C, C), lambda b, j: (0, 0)),
            pl.BlockSpec((C, C), lambda b, j: (0, 0)),
            pl.BlockSpec((C, 1), lambda b, j: (0, 0)),
        ],
        out_specs=pl.BlockSpec((1, C, _HB, W), lambda b, j: (b, 0, j, 0)),
        out_shape=jax.ShapeDtypeStruct((B, C, H, W), jnp.float32),
    )(x, y, w1, w2, b2)


# speedup vs baseline: 3.6711x; 1.0909x over previous
import jax
import jax.numpy as jnp
from jax.experimental import pallas as pl

_HB = 64


def _conv1x1_kernel(x_ref, y_ref, w1_ref, w2_ref, b_ref, o_ref):
    w1 = w1_ref[...]
    w2 = w2_ref[...]
    b = b_ref[...]
    C = w1.shape[0]
    xs = x_ref[0].reshape(C, _HB * 128)
    ys = y_ref[0].reshape(C, _HB * 128)
    acc = jnp.dot(w1, xs, preferred_element_type=jnp.float32)
    acc = acc + jnp.dot(w2, ys, preferred_element_type=jnp.float32)
    o_ref[0] = (acc + b).reshape(C, _HB, 128)


def kernel(x, y, W_proj, b_proj):
    B, C, H, W = x.shape
    w1 = W_proj[:, :C]
    w2 = W_proj[:, C:]
    b2 = b_proj.reshape(C, 1)

    return pl.pallas_call(
        _conv1x1_kernel,
        grid=(B, H // _HB),
        in_specs=[
            pl.BlockSpec((1, C, _HB, W), lambda b, j: (b, 0, j, 0)),
            pl.BlockSpec((1, C, _HB, W), lambda b, j: (b, 0, j, 0)),
            pl.BlockSpec((C, C), lambda b, j: (0, 0)),
            pl.BlockSpec((C, C), lambda b, j: (0, 0)),
            pl.BlockSpec((C, 1), lambda b, j: (0, 0)),
        ],
        out_specs=pl.BlockSpec((1, C, _HB, W), lambda b, j: (b, 0, j, 0)),
        out_shape=jax.ShapeDtypeStruct((B, C, H, W), jnp.float32),
    )(x, y, w1, w2, b2)
